# baseline (speedup 1.0000x reference)
import functools

import jax
import jax.numpy as jnp
from jax.experimental import pallas as pl
from jax.experimental.pallas import tpu as pltpu


def _slab_offsets(D, H, O):
    Hd = H // 2
    G = 8 * Hd
    lay = {}
    r = 0

    def add(name, nrows, ncols, align=8):
        nonlocal r
        if align > 1:
            r = ((r + align - 1) // align) * align
        lay[name] = (r, nrows, ncols)
        r += nrows

    add("w_ih", D, G)
    add("w_hh", 2 * Hd, G)
    add("b_big", 1, G)
    add("b_h", 1, H, align=1)
    add("b_v", 1, D, align=1)
    add("w_w_h", 1, H, align=1)
    add("w_w_v", 1, D, align=1)
    add("w_b", 1, 1, align=1)
    add("b_px", 1, H, align=1)
    add("dec_b", 1, O, align=1)
    add("w_h_f", Hd, H)
    add("w_h_b", Hd, H)
    add("w_v", D, D)
    add("w_p_f", Hd, H)
    add("w_p_b", Hd, H)
    add("w_x", H, H)
    add("dec_w", H, O)
    rows = ((r + 7) // 8) * 8
    return lay, rows


def _atae_kernel(ids_ref, aids_ref,
                 slab_hbm, wemb_hbm, ae_hbm,
                 out_ref,
                 slab, x_sc, asp_sc, xg_sc, outf_sc, outb_sc,
                 wih_r, whh_r, bb_r, sems,
                 *, L, D, H, O, BT, NS, lay):
    Hd = H // 2
    G = 8 * Hd
    SB = BT // NS
    b0 = pl.program_id(0) * BT
    f32 = jnp.float32

    slab_cp = pltpu.make_async_copy(slab_hbm, slab, sems.at[0])
    slab_cp.start()

    for s in range(NS):
        for io in range(SB):
            i = s * SB + io
            pltpu.make_async_copy(ae_hbm.at[pl.ds(aids_ref[b0 + i], 1)],
                                  asp_sc.at[pl.ds(i, 1)],
                                  sems.at[1 + NS + s]).start()
            for t in range(L):
                tok = ids_ref[b0 + i, t]
                pltpu.make_async_copy(
                    wemb_hbm.at[pl.ds(tok, 1)],
                    x_sc.at[pl.ds(s * L * SB + t * SB + io, 1)],
                    sems.at[1 + s]).start(priority=t % 2)

    def ld(name):
        r0, nr, nc = lay[name]
        return slab[r0:r0 + nr, 0:nc]

    slab_cp.wait()
    r_ih, _, _ = lay["w_ih"]
    r_hh, _, _ = lay["w_hh"]
    r_bb, _, _ = lay["b_big"]
    for q in range(4):
        fc = q * 2 * Hd
        bc = q * 2 * Hd + Hd
        wih_r[:, q * Hd:(q + 1) * Hd] = slab[r_ih:r_ih + D, fc:fc + Hd]
        wih_r[:, 4 * Hd + q * Hd:4 * Hd + (q + 1) * Hd] = \
            slab[r_ih:r_ih + D, bc:bc + Hd]
        whh_r[0:Hd, q * Hd:(q + 1) * Hd] = slab[r_hh:r_hh + Hd, fc:fc + Hd]
        whh_r[Hd:2 * Hd, q * Hd:(q + 1) * Hd] = \
            slab[r_hh + Hd:r_hh + 2 * Hd, bc:bc + Hd]
        bb_r[0:1, q * Hd:(q + 1) * Hd] = slab[r_bb:r_bb + 1, fc:fc + Hd]
        bb_r[0:1, 4 * Hd + q * Hd:4 * Hd + (q + 1) * Hd] = \
            slab[r_bb:r_bb + 1, bc:bc + Hd]

    w_h_f = ld("w_h_f")
    w_h_b = ld("w_h_b")
    b_h = ld("b_h")
    w_w_h = ld("w_w_h")
    whh_f = whh_r[0:Hd, :]
    whh_b = whh_r[Hd:2 * Hd, :]

    def compute_sub(s):
        base = s * L * SB
        nrows = L * SB

        pltpu.make_async_copy(wemb_hbm.at[pl.ds(0, nrows)],
                              x_sc.at[pl.ds(base, nrows)],
                              sems.at[1 + s]).wait()
        pltpu.make_async_copy(ae_hbm.at[pl.ds(0, SB)],
                              asp_sc.at[pl.ds(s * SB, SB)],
                              sems.at[1 + NS + s]).wait()

        CH = min(512, nrows)
        for c in range(0, nrows, CH):
            xg_sc[base + c:base + c + CH, :] = (
                jnp.dot(x_sc[base + c:base + c + CH, :], wih_r[...],
                        preferred_element_type=f32)
                + bb_r[0:1, :])

        h_f = jnp.zeros((SB, Hd), f32)
        c_f = jnp.zeros((SB, Hd), f32)
        h_b = jnp.zeros((SB, Hd), f32)
        c_b = jnp.zeros((SB, Hd), f32)
        for t in range(L):
            rf = base + t * SB
            rb = base + (L - 1 - t) * SB
            gf = (xg_sc[rf:rf + SB, 0:4 * Hd]
                  + jnp.dot(h_f, whh_f, preferred_element_type=f32))
            gb = (xg_sc[rb:rb + SB, 4 * Hd:G]
                  + jnp.dot(h_b, whh_b, preferred_element_type=f32))
            sf = jax.nn.sigmoid(gf[:, 0:3 * Hd])
            sb = jax.nn.sigmoid(gb[:, 0:3 * Hd])
            c_f = sf[:, Hd:2 * Hd] * c_f + sf[:, 0:Hd] * jnp.tanh(gf[:, 3 * Hd:])
            c_b = sb[:, Hd:2 * Hd] * c_b + sb[:, 0:Hd] * jnp.tanh(gb[:, 3 * Hd:])
            h_f = sf[:, 2 * Hd:3 * Hd] * jnp.tanh(c_f)
            h_b = sb[:, 2 * Hd:3 * Hd] * jnp.tanh(c_b)
            outf_sc[rf:rf + SB, :] = h_f
            outb_sc[rb:rb + SB, :] = h_b

        hidden = jnp.concatenate([h_f, h_b], axis=1)

        m1_sc = x_sc
        for c in range(0, nrows, CH):
            m1_sc[base + c:base + c + CH, 0:H] = jnp.tanh(
                jnp.dot(outf_sc[base + c:base + c + CH, :], w_h_f,
                        preferred_element_type=f32)
                + jnp.dot(outb_sc[base + c:base + c + CH, :], w_h_b,
                          preferred_element_type=f32)
                + b_h)

        asp = asp_sc[s * SB:(s + 1) * SB, :]
        mv = jnp.tanh(jnp.dot(asp, ld("w_v"), preferred_element_type=f32)
                      + ld("b_v"))
        s_v = jnp.sum(mv * ld("w_w_v"), axis=-1, keepdims=True)
        s_base = s_v + ld("w_b")

        s_t = []
        for t in range(L):
            m1t = m1_sc[base + t * SB:base + (t + 1) * SB, 0:H]
            s_t.append(jnp.sum(m1t * w_w_h, axis=-1, keepdims=True) + s_base)

        m = s_t[0]
        for t in range(1, L):
            m = jnp.maximum(m, s_t[t])
        e_t = [jnp.exp(sc - m) for sc in s_t]
        den = e_t[0]
        for t in range(1, L):
            den = den + e_t[t]
        inv = 1.0 / den

        r_f = jnp.zeros((SB, Hd), f32)
        r_b = jnp.zeros((SB, Hd), f32)
        for t in range(L):
            wa = e_t[t] * inv
            r_f = r_f + wa * outf_sc[base + t * SB:base + (t + 1) * SB, :]
            r_b = r_b + wa * outb_sc[base + t * SB:base + (t + 1) * SB, :]

        r2 = jnp.tanh(
            jnp.dot(r_f, ld("w_p_f"), preferred_element_type=f32)
            + jnp.dot(r_b, ld("w_p_b"), preferred_element_type=f32)
            + jnp.dot(hidden, ld("w_x"), preferred_element_type=f32)
            + ld("b_px"))
        out_ref[s * SB:(s + 1) * SB, :] = (
            jnp.dot(r2, ld("dec_w"), preferred_element_type=f32) + ld("dec_b"))

    for s in range(NS):
        compute_sub(s)


def kernel(slab, word_embed, AE, sentence_ids, aspect_ids):
    B, L = sentence_ids.shape
    D = word_embed.shape[1]
    H = D
    lay, rows = _slab_offsets(D, H, 3)
    O = 3
    BT = 128
    while B % BT:
        BT //= 2
    NS = 2 if BT >= 16 else 1

    kfn = functools.partial(_atae_kernel, L=L, D=D, H=H, O=O, BT=BT, NS=NS,
                            lay=lay)

    return pl.pallas_call(
        kfn,
        out_shape=jax.ShapeDtypeStruct((B, O), jnp.float32),
        grid_spec=pltpu.PrefetchScalarGridSpec(
            num_scalar_prefetch=2,
            grid=(B // BT,),
            in_specs=[
                pl.BlockSpec(memory_space=pl.ANY),
                pl.BlockSpec(memory_space=pl.ANY),
                pl.BlockSpec(memory_space=pl.ANY),
            ],
            out_specs=pl.BlockSpec((BT, O), lambda b, ids, aids: (b, 0)),
            scratch_shapes=[
                pltpu.VMEM((rows, slab.shape[1]), jnp.float32),
                pltpu.VMEM((L * BT, D), jnp.float32),
                pltpu.VMEM((BT, D), jnp.float32),
                pltpu.VMEM((L * BT, 8 * (H // 2)), jnp.float32),
                pltpu.VMEM((L * BT, H // 2), jnp.float32),
                pltpu.VMEM((L * BT, H // 2), jnp.float32),
                pltpu.VMEM((D, 8 * (H // 2)), jnp.float32),
                pltpu.VMEM((H, 4 * (H // 2)), jnp.float32),
                pltpu.VMEM((8, 8 * (H // 2)), jnp.float32),
                pltpu.SemaphoreType.DMA((8,)),
            ],
        ),
        compiler_params=pltpu.CompilerParams(
            dimension_semantics=("parallel",),
            vmem_limit_bytes=56 * 1024 * 1024,
            disable_bounds_checks=True,
        ),
    )(sentence_ids.astype(jnp.int32), aspect_ids.astype(jnp.int32),
      slab, word_embed, AE)
